# Initial kernel scaffold; baseline (speedup 1.0000x reference)
import jax
import jax.numpy as jnp
from jax.experimental import pallas as pl


def kernel(data, table):
    raise NotImplementedError("write your pallas kernel here")


# speedup vs baseline: 1.7965x; 1.7965x over previous
import functools

import jax
import jax.numpy as jnp
from jax import lax
from jax.experimental import pallas as pl
from jax.experimental.pallas import tpu as pltpu
from jax.experimental.pallas import tpu_sc as plsc

VOCAB = 1000000
EMBED_DIM = 64
BATCH = 16384
HIST = 50
B = BATCH * HIST

_NC = 2
_NS = 16
_NW = _NC * _NS
_BPW = B // _NW
_CHUNK = 512
_NCHUNK = _BPW // _CHUNK


def _gather_body(table_hbm, idx_hbm, out_hbm, idx_v, rows_v, sem):
    wid = lax.axis_index("s") * _NC + lax.axis_index("c")
    base = wid * _BPW

    def body(i, carry):
        off = base + i * _CHUNK
        pltpu.sync_copy(idx_hbm.at[pl.ds(off, _CHUNK)], idx_v)
        pltpu.async_copy(table_hbm.at[idx_v], rows_v, sem).wait()
        pltpu.sync_copy(rows_v, out_hbm.at[pl.ds(off, _CHUNK)])
        return carry

    lax.fori_loop(0, _NCHUNK, body, 0)


@functools.partial(jax.jit, static_argnames=())
def _gather(table, idx):
    mesh = plsc.VectorSubcoreMesh(core_axis_name="c", subcore_axis_name="s")
    run = functools.partial(
        pl.kernel,
        mesh=mesh,
        out_type=jax.ShapeDtypeStruct((B, EMBED_DIM), jnp.float32),
        scratch_types=[
            pltpu.VMEM((_CHUNK,), jnp.int32),
            pltpu.VMEM((_CHUNK, EMBED_DIM), jnp.float32),
            pltpu.SemaphoreType.DMA,
        ],
        compiler_params=pltpu.CompilerParams(use_tc_tiling_on_sc=False),
    )(_gather_body)
    return run(table, idx)


def kernel(data, table):
    idx = data.reshape(-1)
    out = _gather(table, idx)
    return out.reshape(BATCH, HIST, EMBED_DIM)


# speedup vs baseline: 1.8669x; 1.0392x over previous
import functools

import jax
import jax.numpy as jnp
from jax import lax
from jax.experimental import pallas as pl
from jax.experimental.pallas import tpu as pltpu
from jax.experimental.pallas import tpu_sc as plsc

VOCAB = 1000000
EMBED_DIM = 64
BATCH = 16384
HIST = 50
B = BATCH * HIST

_NC = 2
_NS = 16
_NW = _NC * _NS
_BPW = B // _NW
_CHUNK = 512
_NCHUNK = _BPW // _CHUNK


def _gather_body(table_hbm, idx_hbm, out_hbm,
                 iv0, iv1, rows0, rows1, g0, g1, w0, w1):
    wid = lax.axis_index("s") * _NC + lax.axis_index("c")
    base = wid * _BPW
    ivs, rows, gsem, wsem = (iv0, iv1), (rows0, rows1), (g0, g1), (w0, w1)

    def start_gather(i, b):
        off = base + i * _CHUNK
        pltpu.sync_copy(idx_hbm.at[pl.ds(off, _CHUNK)], ivs[b])
        pltpu.async_copy(table_hbm.at[ivs[b]], rows[b], gsem[b])

    def wait_gather(b):
        pltpu.make_async_copy(table_hbm.at[ivs[b]], rows[b], gsem[b]).wait()

    def start_write(i, b):
        off = base + i * _CHUNK
        pltpu.async_copy(rows[b], out_hbm.at[pl.ds(off, _CHUNK)], wsem[b])

    def wait_write(i, b):
        off = base + i * _CHUNK
        pltpu.make_async_copy(rows[b], out_hbm.at[pl.ds(off, _CHUNK)],
                              wsem[b]).wait()

    start_gather(0, 0)
    start_gather(1, 1)
    wait_gather(0)
    start_write(0, 0)

    def step(j, carry):
        for b in range(2):
            i = 2 * j + b
            wait_write(i - 2, b)
            start_gather(i, b)
            wait_gather(1 - b)
            start_write(i - 1, 1 - b)
        return carry

    lax.fori_loop(1, _NCHUNK // 2, step, 0)

    wait_gather(1)
    start_write(_NCHUNK - 1, 1)
    wait_write(_NCHUNK - 2, 0)
    wait_write(_NCHUNK - 1, 1)


@jax.jit
def _gather(table, idx):
    mesh = plsc.VectorSubcoreMesh(core_axis_name="c", subcore_axis_name="s")
    run = functools.partial(
        pl.kernel,
        mesh=mesh,
        out_type=jax.ShapeDtypeStruct((B, EMBED_DIM), jnp.float32),
        scratch_types=[
            pltpu.VMEM((_CHUNK,), jnp.int32),
            pltpu.VMEM((_CHUNK,), jnp.int32),
            pltpu.VMEM((_CHUNK, EMBED_DIM), jnp.float32),
            pltpu.VMEM((_CHUNK, EMBED_DIM), jnp.float32),
            pltpu.SemaphoreType.DMA,
            pltpu.SemaphoreType.DMA,
            pltpu.SemaphoreType.DMA,
            pltpu.SemaphoreType.DMA,
        ],
        compiler_params=pltpu.CompilerParams(use_tc_tiling_on_sc=False),
    )(_gather_body)
    return run(table, idx)


def kernel(data, table):
    idx = data.reshape(-1)
    out = _gather(table, idx)
    return out.reshape(BATCH, HIST, EMBED_DIM)
